# baseline (speedup 1.0000x reference)
import jax
import jax.numpy as jnp
from jax import lax
from jax.experimental import pallas as pl
from jax.experimental.pallas import tpu as pltpu
from jax.experimental.pallas import tpu_sc as plsc

N_NODES = 10000
N_EDGES = 320000
N_PAD = 10240
TRASH_ROW = N_NODES

NC, NS = 2, 16
NW = NC * NS
CH = 128
KT = 160
K0, K1 = 80, 80
HC = 16
EPW = KT * CH // 2
E_PAD = NS * KT * CH
ACC_ROWS = 10112
ROWS_PER_SUB = ACC_ROWS // NS
CPB = 128
F = 128

BN_SCALE = float((1.0 + 1e-5) ** -0.5)
SLOPE = 0.01
BLK = 1024

_f32 = jnp.float32
_mesh = plsc.VectorSubcoreMesh(core_axis_name="c", subcore_axis_name="s")


def _deg_body(dst_hbm, out_hbm, dst_v, hist_v):
    c = lax.axis_index("c")
    s = lax.axis_index("s")
    w = c * NS + s

    def _z(i, carry):
        hist_v[pl.ds(i * 16, 16)] = jnp.zeros((16,), _f32)
        return carry
    lax.fori_loop(0, N_PAD // 16, _z, 0)

    pltpu.sync_copy(dst_hbm.at[w], dst_v)
    ones = jnp.ones((16,), _f32)

    def _acc(i, carry):
        idx = dst_v[pl.ds(i * 16, 16)]
        plsc.addupdate_scatter(hist_v, [idx], ones)
        return carry
    lax.fori_loop(0, EPW // 16, _acc, 0)

    pltpu.sync_copy(hist_v, out_hbm.at[w])


_deg_call = pl.kernel(
    _deg_body,
    out_type=jax.ShapeDtypeStruct((NW, N_PAD), _f32),
    mesh=_mesh,
    compiler_params=pltpu.CompilerParams(needs_layout_passes=False),
    scratch_types=[
        pltpu.VMEM((EPW,), jnp.int32),
        pltpu.VMEM((N_PAD,), _f32),
    ],
)


def _agg_body(hp_hbm, src_hbm, dst_hbm, out_hbm,
              src_v, dst_v, rows_v, acc_sh, sem0, sem1, isem):
    c = lax.axis_index("c")
    s = lax.axis_index("s")
    base = s * ROWS_PER_SUB
    io_chunks = [(k * CPB, CPB) for k in range(4)] + [(4 * CPB, ROWS_PER_SUB - 4 * CPB)]

    for off, ln_io in io_chunks:
        pltpu.sync_copy(hp_hbm.at[pl.ds(base + off, ln_io)], rows_v.at[0, pl.ds(0, ln_io)])
        pltpu.sync_copy(rows_v.at[0, pl.ds(0, ln_io)], acc_sh.at[pl.ds(base + off, ln_io)])
    plsc.subcore_barrier()

    base_k = c * K0
    ln = jnp.where(c == 0, K0, K1)
    nh = ln // HC

    pltpu.sync_copy(src_hbm.at[s, pl.ds(base_k, HC)], src_v.at[0])
    pltpu.sync_copy(dst_hbm.at[s, pl.ds(base_k, HC)], dst_v.at[0])

    @pl.when(nh > 1)
    def _():
        pltpu.async_copy(src_hbm.at[s, pl.ds(base_k + HC, HC)], src_v.at[1], isem)
        pltpu.async_copy(dst_hbm.at[s, pl.ds(base_k + HC, HC)], dst_v.at[1], isem)

    pltpu.async_copy(hp_hbm.at[src_v.at[0, 0]], rows_v.at[0], sem0)
    pltpu.async_copy(hp_hbm.at[src_v.at[0, 1]], rows_v.at[1], sem1)

    def _step(t, carry):
        j0 = t * 2
        j1 = j0 + 1
        j2 = j0 + 2
        hh2 = j2 // HC
        p2 = hh2 % 2
        crossing = jnp.logical_and(j2 < ln, j2 % HC == 0)

        pltpu.make_async_copy(hp_hbm.at[src_v.at[0, 0]], rows_v.at[0], sem0).wait()
        pltpu.sync_copy(rows_v.at[0], acc_sh.at[dst_v.at[j0 // HC % 2, j0 % HC]], add=True)

        @pl.when(crossing)
        def _():
            pltpu.make_async_copy(src_hbm.at[s, pl.ds(base_k, HC)], src_v.at[0], isem).wait()
            pltpu.make_async_copy(dst_hbm.at[s, pl.ds(base_k, HC)], dst_v.at[0], isem).wait()

        @pl.when(j2 < ln)
        def _():
            pltpu.async_copy(hp_hbm.at[src_v.at[p2, j2 % HC]], rows_v.at[0], sem0)

        pltpu.make_async_copy(hp_hbm.at[src_v.at[0, 0]], rows_v.at[1], sem1).wait()
        pltpu.sync_copy(rows_v.at[1], acc_sh.at[dst_v.at[j1 // HC % 2, j1 % HC]], add=True)

        @pl.when(jnp.logical_and(crossing, hh2 + 1 < nh))
        def _():
            koff = base_k + (hh2 + 1) * HC
            pltpu.async_copy(src_hbm.at[s, pl.ds(koff, HC)], src_v.at[(hh2 + 1) % 2], isem)
            pltpu.async_copy(dst_hbm.at[s, pl.ds(koff, HC)], dst_v.at[(hh2 + 1) % 2], isem)

        @pl.when(j1 + 2 < ln)
        def _():
            j3 = j1 + 2
            pltpu.async_copy(hp_hbm.at[src_v.at[j3 // HC % 2, j3 % HC]], rows_v.at[1], sem1)
        return carry
    lax.fori_loop(0, ln // 2, _step, 0)
    plsc.subcore_barrier()

    for off, ln_io in io_chunks:
        pltpu.sync_copy(acc_sh.at[pl.ds(base + off, ln_io)], rows_v.at[0, pl.ds(0, ln_io)])
        pltpu.sync_copy(rows_v.at[0, pl.ds(0, ln_io)], out_hbm.at[c, pl.ds(base + off, ln_io)])


_agg_call = pl.kernel(
    _agg_body,
    out_type=jax.ShapeDtypeStruct((NC, N_PAD, F), _f32),
    mesh=_mesh,
    scratch_types=[
        pltpu.VMEM((2, HC, CH), jnp.int32),
        pltpu.VMEM((2, HC, CH), jnp.int32),
        pltpu.VMEM((2, CH, F), _f32),
        pltpu.VMEM_SHARED((ACC_ROWS, F), _f32),
        pltpu.SemaphoreType.DMA,
        pltpu.SemaphoreType.DMA,
        pltpu.SemaphoreType.DMA,
    ],
)


def _lrelu(v):
    return jnp.where(v >= 0, v, SLOPE * v)


def _k0_body(degs_ref, x_ref, w_ref, dinv_ref, hp_ref):
    deg = jnp.sum(degs_ref[...], axis=1, keepdims=True) + 2.0
    dinv = lax.rsqrt(deg)
    dinv_ref[...] = dinv
    h = jnp.dot(x_ref[...], w_ref[...], preferred_element_type=_f32)
    hp_ref[...] = h * dinv


def _k0(degT, x_p, W1p):
    return pl.pallas_call(
        _k0_body,
        grid=(N_PAD // BLK,),
        in_specs=[
            pl.BlockSpec((BLK, NW), lambda i: (i, 0)),
            pl.BlockSpec((BLK, F), lambda i: (i, 0)),
            pl.BlockSpec((F, F), lambda i: (0, 0)),
        ],
        out_specs=[
            pl.BlockSpec((BLK, 1), lambda i: (i, 0)),
            pl.BlockSpec((BLK, F), lambda i: (i, 0)),
        ],
        out_shape=[
            jax.ShapeDtypeStruct((N_PAD, 1), _f32),
            jax.ShapeDtypeStruct((N_PAD, F), _f32),
        ],
    )(degT, x_p, W1p)


def _mid_body(accA_ref, accB_ref, dinv_ref, b_ref, g_ref, be_ref, w_ref, out_ref):
    agg = (accA_ref[...] + accB_ref[...]) * dinv_ref[...]
    z = (agg + b_ref[...]) * (g_ref[...] * BN_SCALE) + be_ref[...]
    z = _lrelu(z)
    out_ref[...] = jnp.dot(z, w_ref[...], preferred_element_type=_f32) * dinv_ref[...]


def _mid(accA, accB, dinv, bp, gp, bep, Wp):
    vec = pl.BlockSpec((1, F), lambda i: (0, 0))
    return pl.pallas_call(
        _mid_body,
        grid=(N_PAD // BLK,),
        in_specs=[
            pl.BlockSpec((BLK, F), lambda i: (i, 0)),
            pl.BlockSpec((BLK, F), lambda i: (i, 0)),
            pl.BlockSpec((BLK, 1), lambda i: (i, 0)),
            vec, vec, vec,
            pl.BlockSpec((F, F), lambda i: (0, 0)),
        ],
        out_specs=pl.BlockSpec((BLK, F), lambda i: (i, 0)),
        out_shape=jax.ShapeDtypeStruct((N_PAD, F), _f32),
    )(accA, accB, dinv, bp, gp, bep, Wp)


def _fin_body(accA_ref, accB_ref, dinv_ref, b_ref, g_ref, be_ref,
              fw1_ref, fb1_ref, fw2_ref, fb2_ref, fw3_ref, fb3_ref,
              y_ref, o_ref):
    agg = (accA_ref[...] + accB_ref[...]) * dinv_ref[...]
    z = (agg + b_ref[...]) * (g_ref[...] * BN_SCALE) + be_ref[...]
    z = _lrelu(z)
    y_ref[...] = z
    o1 = _lrelu(jnp.dot(z, fw1_ref[...], preferred_element_type=_f32) + fb1_ref[...])
    o2 = _lrelu(jnp.dot(o1, fw2_ref[...], preferred_element_type=_f32) + fb2_ref[...])
    o_ref[...] = jnp.dot(o2, fw3_ref[...], preferred_element_type=_f32) + fb3_ref[...]


def _fin(accA, accB, dinv, b, g, be, fW1, fb1, fW2, fb2, fW3, fb3):
    vec = pl.BlockSpec((1, F), lambda i: (0, 0))
    full = lambda a: pl.BlockSpec(a.shape, lambda i: (0, 0))
    return pl.pallas_call(
        _fin_body,
        grid=(N_PAD // BLK,),
        in_specs=[
            pl.BlockSpec((BLK, F), lambda i: (i, 0)),
            pl.BlockSpec((BLK, F), lambda i: (i, 0)),
            pl.BlockSpec((BLK, 1), lambda i: (i, 0)),
            vec, vec, vec,
            full(fW1), pl.BlockSpec((1, 64), lambda i: (0, 0)),
            full(fW2), pl.BlockSpec((1, 32), lambda i: (0, 0)),
            full(fW3), pl.BlockSpec((1, 1), lambda i: (0, 0)),
        ],
        out_specs=[
            pl.BlockSpec((BLK, F), lambda i: (i, 0)),
            pl.BlockSpec((BLK, 1), lambda i: (i, 0)),
        ],
        out_shape=[
            jax.ShapeDtypeStruct((N_PAD, F), _f32),
            jax.ShapeDtypeStruct((N_PAD, 1), _f32),
        ],
    )(accA, accB, dinv, b.reshape(1, F), g.reshape(1, F), be.reshape(1, F),
      fW1, fb1.reshape(1, 64), fW2, fb2.reshape(1, 32), fW3, fb3.reshape(1, 1))


def _padw(W):
    fi, fo = W.shape
    return jnp.pad(W, ((0, F - fi), (0, F - fo)))


def _padv(v):
    return jnp.pad(v, (0, F - v.shape[0])).reshape(1, F)


def kernel(x, edge_index, W1, b1, g1, be1, W2, b2, g2, be2, W3, b3, g3, be3,
           fW1, fb1, fW2, fb2, fW3, fb3):
    src = edge_index[0].astype(jnp.int32)
    dst = edge_index[1].astype(jnp.int32)
    pad = E_PAD - N_EDGES
    src_p = jnp.concatenate([src, jnp.zeros((pad,), jnp.int32)]).reshape(NS, KT, CH)
    trash = TRASH_ROW + (jnp.arange(pad, dtype=jnp.int32) % (ACC_ROWS - N_NODES))
    dst_p = jnp.concatenate([dst, trash]).reshape(NS, KT, CH)
    dst_flat = dst_p.reshape(NW, EPW)
    x_p = jnp.pad(x, ((0, N_PAD - N_NODES), (0, 0)))

    degs = _deg_call(dst_flat)
    degT = degs.T
    dinv, h1p = _k0(degT, x_p, _padw(W1))

    acc1 = _agg_call(h1p, src_p, dst_p)
    h2p = _mid(acc1[0], acc1[1], dinv, _padv(b1), _padv(g1), _padv(be1), _padw(W2))
    acc2 = _agg_call(h2p, src_p, dst_p)
    h3p = _mid(acc2[0], acc2[1], dinv, _padv(b2), _padv(g2), _padv(be2), _padw(W3))
    acc3 = _agg_call(h3p, src_p, dst_p)
    y_p, o_p = _fin(acc3[0], acc3[1], dinv, b3, g3, be3,
                    fW1, fb1, fW2, fb2, fW3, fb3)
    return y_p[:N_NODES], o_p[:N_NODES]


# speedup vs baseline: 2.3328x; 2.3328x over previous
import jax
import jax.numpy as jnp
from jax import lax
from jax.experimental import pallas as pl
from jax.experimental.pallas import tpu as pltpu
from jax.experimental.pallas import tpu_sc as plsc

N_NODES = 10000
N_EDGES = 320000
N_PAD = 10240
TRASH_ROW = N_NODES

NC, NS = 2, 16
NW = NC * NS
CH = 128
KT = 160
K0, K1 = 128, 32
HC = 32
EPW = KT * CH // 2
E_PAD = NS * KT * CH
ACC_ROWS = 10112
ROWS_PER_SUB = ACC_ROWS // NS
CPB = 128
F = 128

BN_SCALE = float((1.0 + 1e-5) ** -0.5)
SLOPE = 0.01
BLK = 1024

_f32 = jnp.float32
_mesh = plsc.VectorSubcoreMesh(core_axis_name="c", subcore_axis_name="s")


def _deg_body(dst_hbm, out_hbm, dst_v, hist_v):
    c = lax.axis_index("c")
    s = lax.axis_index("s")
    w = c * NS + s

    def _z(i, carry):
        hist_v[pl.ds(i * 16, 16)] = jnp.zeros((16,), _f32)
        return carry
    lax.fori_loop(0, N_PAD // 16, _z, 0)

    pltpu.sync_copy(dst_hbm.at[w], dst_v)
    ones = jnp.ones((16,), _f32)

    def _acc(i, carry):
        idx = dst_v[pl.ds(i * 16, 16)]
        plsc.addupdate_scatter(hist_v, [idx], ones)
        return carry
    lax.fori_loop(0, EPW // 16, _acc, 0)

    pltpu.sync_copy(hist_v, out_hbm.at[w])


_deg_call = pl.kernel(
    _deg_body,
    out_type=jax.ShapeDtypeStruct((NW, N_PAD), _f32),
    mesh=_mesh,
    compiler_params=pltpu.CompilerParams(needs_layout_passes=False),
    scratch_types=[
        pltpu.VMEM((EPW,), jnp.int32),
        pltpu.VMEM((N_PAD,), _f32),
    ],
)


def _agg_body(hp_hbm, src_hbm, dst_hbm, out_hbm,
              src_v, dst_v, rows_v, acc_sh, sem0, sem1, isem):
    c = lax.axis_index("c")
    s = lax.axis_index("s")
    base = s * ROWS_PER_SUB
    io_chunks = [(k * CPB, CPB) for k in range(4)] + [(4 * CPB, ROWS_PER_SUB - 4 * CPB)]

    for off, ln_io in io_chunks:
        pltpu.sync_copy(hp_hbm.at[pl.ds(base + off, ln_io)], rows_v.at[0, pl.ds(0, ln_io)])
        pltpu.sync_copy(rows_v.at[0, pl.ds(0, ln_io)], acc_sh.at[pl.ds(base + off, ln_io)])
    plsc.subcore_barrier()

    base_k = c * K0
    ln = jnp.where(c == 0, K0, K1)
    nh = ln // HC

    pltpu.sync_copy(src_hbm.at[s, pl.ds(base_k, HC)], src_v.at[0])
    pltpu.sync_copy(dst_hbm.at[s, pl.ds(base_k, HC)], dst_v.at[0])

    @pl.when(nh > 1)
    def _():
        pltpu.async_copy(src_hbm.at[s, pl.ds(base_k + HC, HC)], src_v.at[1], isem)
        pltpu.async_copy(dst_hbm.at[s, pl.ds(base_k + HC, HC)], dst_v.at[1], isem)

    pltpu.async_copy(hp_hbm.at[src_v.at[0, 0]], rows_v.at[0], sem0)
    pltpu.async_copy(hp_hbm.at[src_v.at[0, 1]], rows_v.at[1], sem1)

    def _step(t, carry):
        j0 = t * 2
        j1 = j0 + 1
        j2 = j0 + 2
        hh2 = j2 // HC
        p2 = hh2 % 2
        crossing = jnp.logical_and(j2 < ln, j2 % HC == 0)

        pltpu.make_async_copy(hp_hbm.at[src_v.at[0, 0]], rows_v.at[0], sem0).wait()
        pltpu.sync_copy(rows_v.at[0], acc_sh.at[dst_v.at[j0 // HC % 2, j0 % HC]], add=True)

        @pl.when(crossing)
        def _():
            pltpu.make_async_copy(src_hbm.at[s, pl.ds(base_k, HC)], src_v.at[0], isem).wait()
            pltpu.make_async_copy(dst_hbm.at[s, pl.ds(base_k, HC)], dst_v.at[0], isem).wait()

        @pl.when(j2 < ln)
        def _():
            pltpu.async_copy(hp_hbm.at[src_v.at[p2, j2 % HC]], rows_v.at[0], sem0)

        pltpu.make_async_copy(hp_hbm.at[src_v.at[0, 0]], rows_v.at[1], sem1).wait()
        pltpu.sync_copy(rows_v.at[1], acc_sh.at[dst_v.at[j1 // HC % 2, j1 % HC]], add=True)

        @pl.when(jnp.logical_and(crossing, hh2 + 1 < nh))
        def _():
            koff = base_k + (hh2 + 1) * HC
            pltpu.async_copy(src_hbm.at[s, pl.ds(koff, HC)], src_v.at[(hh2 + 1) % 2], isem)
            pltpu.async_copy(dst_hbm.at[s, pl.ds(koff, HC)], dst_v.at[(hh2 + 1) % 2], isem)

        @pl.when(j1 + 2 < ln)
        def _():
            j3 = j1 + 2
            pltpu.async_copy(hp_hbm.at[src_v.at[j3 // HC % 2, j3 % HC]], rows_v.at[1], sem1)
        return carry
    lax.fori_loop(0, ln // 2, _step, 0)
    plsc.subcore_barrier()

    for off, ln_io in io_chunks:
        pltpu.sync_copy(acc_sh.at[pl.ds(base + off, ln_io)], rows_v.at[0, pl.ds(0, ln_io)])
        pltpu.sync_copy(rows_v.at[0, pl.ds(0, ln_io)], out_hbm.at[c, pl.ds(base + off, ln_io)])


_agg_call = pl.kernel(
    _agg_body,
    out_type=jax.ShapeDtypeStruct((NC, N_PAD, F), _f32),
    mesh=_mesh,
    scratch_types=[
        pltpu.VMEM((2, HC, CH), jnp.int32),
        pltpu.VMEM((2, HC, CH), jnp.int32),
        pltpu.VMEM((2, CH, F), _f32),
        pltpu.VMEM_SHARED((ACC_ROWS, F), _f32),
        pltpu.SemaphoreType.DMA,
        pltpu.SemaphoreType.DMA,
        pltpu.SemaphoreType.DMA,
    ],
)


def _lrelu(v):
    return jnp.where(v >= 0, v, SLOPE * v)


def _k0_body(degs_ref, x_ref, w_ref, dinv_ref, hp_ref):
    deg = jnp.sum(degs_ref[...], axis=1, keepdims=True) + 2.0
    rows = pl.program_id(0) * BLK + lax.broadcasted_iota(jnp.int32, (BLK, 1), 0)
    dinv = jnp.where(rows < N_NODES, lax.rsqrt(deg), 0.0)
    dinv_ref[...] = dinv
    h = jnp.dot(x_ref[...], w_ref[...], preferred_element_type=_f32)
    hp_ref[...] = h * dinv


def _k0(degT, x_p, W1p):
    return pl.pallas_call(
        _k0_body,
        grid=(N_PAD // BLK,),
        in_specs=[
            pl.BlockSpec((BLK, NW), lambda i: (i, 0)),
            pl.BlockSpec((BLK, F), lambda i: (i, 0)),
            pl.BlockSpec((F, F), lambda i: (0, 0)),
        ],
        out_specs=[
            pl.BlockSpec((BLK, 1), lambda i: (i, 0)),
            pl.BlockSpec((BLK, F), lambda i: (i, 0)),
        ],
        out_shape=[
            jax.ShapeDtypeStruct((N_PAD, 1), _f32),
            jax.ShapeDtypeStruct((N_PAD, F), _f32),
        ],
    )(degT, x_p, W1p)


def _mid_body(accA_ref, accB_ref, dinv_ref, b_ref, g_ref, be_ref, w_ref, out_ref):
    agg = (accA_ref[...] + accB_ref[...]) * dinv_ref[...]
    z = (agg + b_ref[...]) * (g_ref[...] * BN_SCALE) + be_ref[...]
    z = _lrelu(z)
    out_ref[...] = jnp.dot(z, w_ref[...], preferred_element_type=_f32) * dinv_ref[...]


def _mid(accA, accB, dinv, bp, gp, bep, Wp):
    vec = pl.BlockSpec((1, F), lambda i: (0, 0))
    return pl.pallas_call(
        _mid_body,
        grid=(N_PAD // BLK,),
        in_specs=[
            pl.BlockSpec((BLK, F), lambda i: (i, 0)),
            pl.BlockSpec((BLK, F), lambda i: (i, 0)),
            pl.BlockSpec((BLK, 1), lambda i: (i, 0)),
            vec, vec, vec,
            pl.BlockSpec((F, F), lambda i: (0, 0)),
        ],
        out_specs=pl.BlockSpec((BLK, F), lambda i: (i, 0)),
        out_shape=jax.ShapeDtypeStruct((N_PAD, F), _f32),
    )(accA, accB, dinv, bp, gp, bep, Wp)


def _fin_body(accA_ref, accB_ref, dinv_ref, b_ref, g_ref, be_ref,
              fw1_ref, fb1_ref, fw2_ref, fb2_ref, fw3_ref, fb3_ref,
              y_ref, o_ref):
    agg = (accA_ref[...] + accB_ref[...]) * dinv_ref[...]
    z = (agg + b_ref[...]) * (g_ref[...] * BN_SCALE) + be_ref[...]
    z = _lrelu(z)
    y_ref[...] = z
    o1 = _lrelu(jnp.dot(z, fw1_ref[...], preferred_element_type=_f32) + fb1_ref[...])
    o2 = _lrelu(jnp.dot(o1, fw2_ref[...], preferred_element_type=_f32) + fb2_ref[...])
    o_ref[...] = jnp.dot(o2, fw3_ref[...], preferred_element_type=_f32) + fb3_ref[...]


def _fin(accA, accB, dinv, b, g, be, fW1, fb1, fW2, fb2, fW3, fb3):
    vec = pl.BlockSpec((1, F), lambda i: (0, 0))
    full = lambda a: pl.BlockSpec(a.shape, lambda i: (0, 0))
    return pl.pallas_call(
        _fin_body,
        grid=(N_PAD // BLK,),
        in_specs=[
            pl.BlockSpec((BLK, F), lambda i: (i, 0)),
            pl.BlockSpec((BLK, F), lambda i: (i, 0)),
            pl.BlockSpec((BLK, 1), lambda i: (i, 0)),
            vec, vec, vec,
            full(fW1), pl.BlockSpec((1, 64), lambda i: (0, 0)),
            full(fW2), pl.BlockSpec((1, 32), lambda i: (0, 0)),
            full(fW3), pl.BlockSpec((1, 1), lambda i: (0, 0)),
        ],
        out_specs=[
            pl.BlockSpec((BLK, F), lambda i: (i, 0)),
            pl.BlockSpec((BLK, 1), lambda i: (i, 0)),
        ],
        out_shape=[
            jax.ShapeDtypeStruct((N_PAD, F), _f32),
            jax.ShapeDtypeStruct((N_PAD, 1), _f32),
        ],
    )(accA, accB, dinv, b.reshape(1, F), g.reshape(1, F), be.reshape(1, F),
      fW1, fb1.reshape(1, 64), fW2, fb2.reshape(1, 32), fW3, fb3.reshape(1, 1))


def _padw(W):
    fi, fo = W.shape
    return jnp.pad(W, ((0, F - fi), (0, F - fo)))


def _padv(v):
    return jnp.pad(v, (0, F - v.shape[0])).reshape(1, F)


def kernel(x, edge_index, W1, b1, g1, be1, W2, b2, g2, be2, W3, b3, g3, be3,
           fW1, fb1, fW2, fb2, fW3, fb3):
    src = edge_index[0].astype(jnp.int32)
    dst = edge_index[1].astype(jnp.int32)
    pad = E_PAD - N_EDGES
    ar = jnp.arange(pad, dtype=jnp.int32)
    pad_src = TRASH_ROW + ar % (ACC_ROWS - N_NODES)
    pad_dst = ar % ACC_ROWS
    src_p = jnp.concatenate([src, pad_src]).reshape(NS, KT, CH)
    dst_p = jnp.concatenate([dst, pad_dst]).reshape(NS, KT, CH)
    dst_flat = jnp.concatenate([dst, pad_src]).reshape(NW, EPW)
    x_p = jnp.pad(x, ((0, N_PAD - N_NODES), (0, 0)))

    degs = _deg_call(dst_flat)
    degT = degs.T
    dinv, h1p = _k0(degT, x_p, _padw(W1))

    acc1 = _agg_call(h1p, src_p, dst_p)
    h2p = _mid(acc1[0], acc1[1], dinv, _padv(b1), _padv(g1), _padv(be1), _padw(W2))
    acc2 = _agg_call(h2p, src_p, dst_p)
    h3p = _mid(acc2[0], acc2[1], dinv, _padv(b2), _padv(g2), _padv(be2), _padw(W3))
    acc3 = _agg_call(h3p, src_p, dst_p)
    y_p, o_p = _fin(acc3[0], acc3[1], dinv, b3, g3, be3,
                    fW1, fb1, fW2, fb2, fW3, fb3)
    return y_p[:N_NODES], o_p[:N_NODES]


# speedup vs baseline: 3.2102x; 1.3761x over previous
import jax
import jax.numpy as jnp
from jax import lax
from jax.experimental import pallas as pl
from jax.experimental.pallas import tpu as pltpu
from jax.experimental.pallas import tpu_sc as plsc

N_NODES = 10000
N_EDGES = 320000
N_PAD = 10240
TRASH_ROW = N_NODES

NC, NS = 2, 16
NW = NC * NS
CH = 128
KT = 160
K0, K1 = 80, 80
HC = 16
EPW = KT * CH // 2
E_PAD = NS * KT * CH
ACC_ROWS = 10112
ROWS_PER_SUB = ACC_ROWS // NS
CPB = 128
F = 128

BN_SCALE = float((1.0 + 1e-5) ** -0.5)
SLOPE = 0.01
BLK = 1024

_f32 = jnp.float32
_mesh = plsc.VectorSubcoreMesh(core_axis_name="c", subcore_axis_name="s")


def _deg_body(dst_hbm, out_hbm, dst_v, hist_v):
    c = lax.axis_index("c")
    s = lax.axis_index("s")
    w = c * NS + s

    def _z(i, carry):
        hist_v[pl.ds(i * 16, 16)] = jnp.zeros((16,), _f32)
        return carry
    lax.fori_loop(0, N_PAD // 16, _z, 0)

    pltpu.sync_copy(dst_hbm.at[w], dst_v)
    ones = jnp.ones((16,), _f32)

    def _acc(i, carry):
        idx = dst_v[pl.ds(i * 16, 16)]
        plsc.addupdate_scatter(hist_v, [idx], ones)
        return carry
    lax.fori_loop(0, EPW // 16, _acc, 0)

    pltpu.sync_copy(hist_v, out_hbm.at[w])


_deg_call = pl.kernel(
    _deg_body,
    out_type=jax.ShapeDtypeStruct((NW, N_PAD), _f32),
    mesh=_mesh,
    compiler_params=pltpu.CompilerParams(needs_layout_passes=False),
    scratch_types=[
        pltpu.VMEM((EPW,), jnp.int32),
        pltpu.VMEM((N_PAD,), _f32),
    ],
)


def _agg_body(hp_hbm, src_hbm, dst_hbm, out_hbm,
              src_v, dst_v, rows_v, acc_sh, sem0, sem1, isem):
    c = lax.axis_index("c")
    s = lax.axis_index("s")
    base = s * ROWS_PER_SUB
    io_chunks = [(k * CPB, CPB) for k in range(4)] + [(4 * CPB, ROWS_PER_SUB - 4 * CPB)]

    for off, ln_io in io_chunks:
        pltpu.sync_copy(hp_hbm.at[pl.ds(base + off, ln_io)], rows_v.at[0, pl.ds(0, ln_io)])
        pltpu.sync_copy(rows_v.at[0, pl.ds(0, ln_io)], acc_sh.at[pl.ds(base + off, ln_io)])
    plsc.subcore_barrier()

    base_k = c * K0
    ln = jnp.where(c == 0, K0, K1)
    nh = ln // HC

    pltpu.sync_copy(src_hbm.at[s, pl.ds(base_k, HC)], src_v.at[0])
    pltpu.sync_copy(dst_hbm.at[s, pl.ds(base_k, HC)], dst_v.at[0])

    @pl.when(nh > 1)
    def _():
        pltpu.async_copy(src_hbm.at[s, pl.ds(base_k + HC, HC)], src_v.at[1], isem)
        pltpu.async_copy(dst_hbm.at[s, pl.ds(base_k + HC, HC)], dst_v.at[1], isem)

    pltpu.async_copy(hp_hbm.at[src_v.at[0, 0]], rows_v.at[0], sem0)
    pltpu.async_copy(hp_hbm.at[src_v.at[0, 1]], rows_v.at[1], sem1)

    def _step(t, carry):
        j0 = t * 2
        j1 = j0 + 1
        j2 = j0 + 2
        hh2 = j2 // HC
        p2 = hh2 % 2
        crossing = jnp.logical_and(j2 < ln, j2 % HC == 0)

        pltpu.make_async_copy(hp_hbm.at[src_v.at[0, 0]], rows_v.at[0], sem0).wait()
        pltpu.sync_copy(rows_v.at[0], acc_sh.at[dst_v.at[j0 // HC % 2, j0 % HC]], add=True)

        @pl.when(crossing)
        def _():
            pltpu.make_async_copy(src_hbm.at[s, pl.ds(base_k, HC)], src_v.at[0], isem).wait()
            pltpu.make_async_copy(dst_hbm.at[s, pl.ds(base_k, HC)], dst_v.at[0], isem).wait()

        @pl.when(j2 < ln)
        def _():
            pltpu.async_copy(hp_hbm.at[src_v.at[p2, j2 % HC]], rows_v.at[0], sem0)

        pltpu.make_async_copy(hp_hbm.at[src_v.at[0, 0]], rows_v.at[1], sem1).wait()
        pltpu.sync_copy(rows_v.at[1], acc_sh.at[dst_v.at[j1 // HC % 2, j1 % HC]], add=True)

        @pl.when(jnp.logical_and(crossing, hh2 + 1 < nh))
        def _():
            koff = base_k + (hh2 + 1) * HC
            pltpu.async_copy(src_hbm.at[s, pl.ds(koff, HC)], src_v.at[(hh2 + 1) % 2], isem)
            pltpu.async_copy(dst_hbm.at[s, pl.ds(koff, HC)], dst_v.at[(hh2 + 1) % 2], isem)

        @pl.when(j1 + 2 < ln)
        def _():
            j3 = j1 + 2
            pltpu.async_copy(hp_hbm.at[src_v.at[j3 // HC % 2, j3 % HC]], rows_v.at[1], sem1)
        return carry
    lax.fori_loop(0, ln // 2, _step, 0)
    plsc.subcore_barrier()

    for off, ln_io in io_chunks:
        pltpu.sync_copy(acc_sh.at[pl.ds(base + off, ln_io)], rows_v.at[0, pl.ds(0, ln_io)])
        pltpu.sync_copy(rows_v.at[0, pl.ds(0, ln_io)], out_hbm.at[c, pl.ds(base + off, ln_io)])


_agg_call = pl.kernel(
    _agg_body,
    out_type=jax.ShapeDtypeStruct((NC, N_PAD, F), _f32),
    mesh=_mesh,
    scratch_types=[
        pltpu.VMEM((2, HC, CH), jnp.int32),
        pltpu.VMEM((2, HC, CH), jnp.int32),
        pltpu.VMEM((2, CH, F), _f32),
        pltpu.VMEM_SHARED((ACC_ROWS, F), _f32),
        pltpu.SemaphoreType.DMA,
        pltpu.SemaphoreType.DMA,
        pltpu.SemaphoreType.DMA,
    ],
)


def _lrelu(v):
    return jnp.where(v >= 0, v, SLOPE * v)


def _k0_body(degs_ref, x_ref, w_ref, dinv_ref, hp_ref):
    deg = jnp.sum(degs_ref[...], axis=1, keepdims=True) + 2.0
    rows = pl.program_id(0) * BLK + lax.broadcasted_iota(jnp.int32, (BLK, 1), 0)
    dinv = jnp.where(rows < N_NODES, lax.rsqrt(deg), 0.0)
    dinv_ref[...] = dinv
    h = jnp.dot(x_ref[...], w_ref[...], preferred_element_type=_f32)
    hp_ref[...] = h * dinv


def _k0(degT, x_p, W1p):
    return pl.pallas_call(
        _k0_body,
        grid=(N_PAD // BLK,),
        in_specs=[
            pl.BlockSpec((BLK, NW), lambda i: (i, 0)),
            pl.BlockSpec((BLK, F), lambda i: (i, 0)),
            pl.BlockSpec((F, F), lambda i: (0, 0)),
        ],
        out_specs=[
            pl.BlockSpec((BLK, 1), lambda i: (i, 0)),
            pl.BlockSpec((BLK, F), lambda i: (i, 0)),
        ],
        out_shape=[
            jax.ShapeDtypeStruct((N_PAD, 1), _f32),
            jax.ShapeDtypeStruct((N_PAD, F), _f32),
        ],
    )(degT, x_p, W1p)


def _mid_body(accA_ref, accB_ref, dinv_ref, b_ref, g_ref, be_ref, w_ref, out_ref):
    agg = (accA_ref[...] + accB_ref[...]) * dinv_ref[...]
    z = (agg + b_ref[...]) * (g_ref[...] * BN_SCALE) + be_ref[...]
    z = _lrelu(z)
    out_ref[...] = jnp.dot(z, w_ref[...], preferred_element_type=_f32) * dinv_ref[...]


def _mid(accA, accB, dinv, bp, gp, bep, Wp):
    vec = pl.BlockSpec((1, F), lambda i: (0, 0))
    return pl.pallas_call(
        _mid_body,
        grid=(N_PAD // BLK,),
        in_specs=[
            pl.BlockSpec((BLK, F), lambda i: (i, 0)),
            pl.BlockSpec((BLK, F), lambda i: (i, 0)),
            pl.BlockSpec((BLK, 1), lambda i: (i, 0)),
            vec, vec, vec,
            pl.BlockSpec((F, F), lambda i: (0, 0)),
        ],
        out_specs=pl.BlockSpec((BLK, F), lambda i: (i, 0)),
        out_shape=jax.ShapeDtypeStruct((N_PAD, F), _f32),
    )(accA, accB, dinv, bp, gp, bep, Wp)


def _fin_body(accA_ref, accB_ref, dinv_ref, b_ref, g_ref, be_ref,
              fw1_ref, fb1_ref, fw2_ref, fb2_ref, fw3_ref, fb3_ref,
              y_ref, o_ref):
    agg = (accA_ref[...] + accB_ref[...]) * dinv_ref[...]
    z = (agg + b_ref[...]) * (g_ref[...] * BN_SCALE) + be_ref[...]
    z = _lrelu(z)
    y_ref[...] = z
    o1 = _lrelu(jnp.dot(z, fw1_ref[...], preferred_element_type=_f32) + fb1_ref[...])
    o2 = _lrelu(jnp.dot(o1, fw2_ref[...], preferred_element_type=_f32) + fb2_ref[...])
    o_ref[...] = jnp.dot(o2, fw3_ref[...], preferred_element_type=_f32) + fb3_ref[...]


def _fin(accA, accB, dinv, b, g, be, fW1, fb1, fW2, fb2, fW3, fb3):
    vec = pl.BlockSpec((1, F), lambda i: (0, 0))
    full = lambda a: pl.BlockSpec(a.shape, lambda i: (0, 0))
    return pl.pallas_call(
        _fin_body,
        grid=(N_PAD // BLK,),
        in_specs=[
            pl.BlockSpec((BLK, F), lambda i: (i, 0)),
            pl.BlockSpec((BLK, F), lambda i: (i, 0)),
            pl.BlockSpec((BLK, 1), lambda i: (i, 0)),
            vec, vec, vec,
            full(fW1), pl.BlockSpec((1, 64), lambda i: (0, 0)),
            full(fW2), pl.BlockSpec((1, 32), lambda i: (0, 0)),
            full(fW3), pl.BlockSpec((1, 1), lambda i: (0, 0)),
        ],
        out_specs=[
            pl.BlockSpec((BLK, F), lambda i: (i, 0)),
            pl.BlockSpec((BLK, 1), lambda i: (i, 0)),
        ],
        out_shape=[
            jax.ShapeDtypeStruct((N_PAD, F), _f32),
            jax.ShapeDtypeStruct((N_PAD, 1), _f32),
        ],
    )(accA, accB, dinv, b.reshape(1, F), g.reshape(1, F), be.reshape(1, F),
      fW1, fb1.reshape(1, 64), fW2, fb2.reshape(1, 32), fW3, fb3.reshape(1, 1))


def _padw(W):
    fi, fo = W.shape
    return jnp.pad(W, ((0, F - fi), (0, F - fo)))


def _padv(v):
    return jnp.pad(v, (0, F - v.shape[0])).reshape(1, F)


def kernel(x, edge_index, W1, b1, g1, be1, W2, b2, g2, be2, W3, b3, g3, be3,
           fW1, fb1, fW2, fb2, fW3, fb3):
    src = edge_index[0].astype(jnp.int32)
    dst = edge_index[1].astype(jnp.int32)
    pad = E_PAD - N_EDGES
    ar = jnp.arange(pad, dtype=jnp.int32)
    pad_src = TRASH_ROW + ar % (ACC_ROWS - N_NODES)
    pad_dst = ar % ACC_ROWS
    src_p = jnp.concatenate([src, pad_src]).reshape(NS, KT, CH)
    dst_p = jnp.concatenate([dst, pad_dst]).reshape(NS, KT, CH)
    dst_flat = jnp.concatenate([dst, pad_src]).reshape(NW, EPW)
    x_p = jnp.pad(x, ((0, N_PAD - N_NODES), (0, 0)))

    degs = _deg_call(dst_flat)
    degT = degs.T
    dinv, h1p = _k0(degT, x_p, _padw(W1))

    acc1 = _agg_call(h1p, src_p, dst_p)
    h2p = _mid(acc1[0], acc1[1], dinv, _padv(b1), _padv(g1), _padv(be1), _padw(W2))
    acc2 = _agg_call(h2p, src_p, dst_p)
    h3p = _mid(acc2[0], acc2[1], dinv, _padv(b2), _padv(g2), _padv(be2), _padw(W3))
    acc3 = _agg_call(h3p, src_p, dst_p)
    y_p, o_p = _fin(acc3[0], acc3[1], dinv, b3, g3, be3,
                    fW1, fb1, fW2, fb2, fW3, fb3)
    return y_p[:N_NODES], o_p[:N_NODES]
